# baseline (speedup 1.0000x reference)
import jax
import jax.numpy as jnp
from jax.experimental import pallas as pl


_BN = 4096


def _body(x_ref, w_ref, b_ref, o_ref):
    i = pl.program_id(0)
    nb = b_ref.shape[1] // _BN
    sl = pl.ds(i * _BN, _BN)

    @pl.when(i < nb)
    def _():
        half = _BN // 2
        for h in range(2):
            hsl = pl.ds(i * _BN + h * half, half)
            o_ref[:, hsl] = jax.lax.dot_general(
                x_ref[...], w_ref[pl.ds(h * half, half), :],
                dimension_numbers=(((1,), (1,)), ((), ())),
                preferred_element_type=jnp.float32,
            ) + b_ref[:, hsl]

    if b_ref.shape[1] % _BN:
        tail = b_ref.shape[1] - nb * _BN
        tsl = pl.ds(nb * _BN, tail)

        @pl.when(i == nb)
        def _():
            o_ref[:, tsl] = jax.lax.dot_general(
                x_ref[...], w_ref[pl.ds(0, tail), :],
                dimension_numbers=(((1,), (1,)), ((), ())),
                preferred_element_type=jnp.float32,
            ) + b_ref[:, tsl]


def kernel(x, tgt, table_w0, table_b0):
    del tgt
    B, I, H = x.shape
    N = table_w0.shape[0]
    x2 = x.reshape(B * I, H)
    b2 = table_b0.reshape(1, N)
    out = pl.pallas_call(
        _body,
        grid=(pl.cdiv(N, _BN),),
        in_specs=[
            pl.BlockSpec((B * I, H), lambda i: (0, 0)),
            pl.BlockSpec((_BN, H), lambda i: (i, 0)),
            pl.BlockSpec((1, N), lambda i: (0, 0)),
        ],
        out_specs=pl.BlockSpec((B * I, N), lambda i: (0, 0)),
        out_shape=jax.ShapeDtypeStruct((B * I, N), jnp.float32),
    )(x2, table_w0, b2)
    return out.reshape(B, I, N)


# speedup vs baseline: 1.0063x; 1.0063x over previous
import jax
import jax.numpy as jnp
from jax.experimental import pallas as pl


_BN = 4608


def _body(x_ref, w_ref, b_ref, o_ref):
    i = pl.program_id(0)
    nb = b_ref.shape[1] // _BN
    sl = pl.ds(i * _BN, _BN)

    @pl.when(i < nb)
    def _():
        o_ref[:, sl] = jax.lax.dot_general(
            x_ref[...], w_ref[...],
            dimension_numbers=(((1,), (1,)), ((), ())),
            preferred_element_type=jnp.float32,
        ) + b_ref[:, sl]

    if b_ref.shape[1] % _BN:
        tail = b_ref.shape[1] - nb * _BN
        tsl = pl.ds(nb * _BN, tail)

        @pl.when(i == nb)
        def _():
            o_ref[:, tsl] = jax.lax.dot_general(
                x_ref[...], w_ref[pl.ds(0, tail), :],
                dimension_numbers=(((1,), (1,)), ((), ())),
                preferred_element_type=jnp.float32,
            ) + b_ref[:, tsl]


def kernel(x, tgt, table_w0, table_b0):
    del tgt
    B, I, H = x.shape
    N = table_w0.shape[0]
    x2 = x.reshape(B * I, H)
    b2 = table_b0.reshape(1, N)
    out = pl.pallas_call(
        _body,
        grid=(pl.cdiv(N, _BN),),
        in_specs=[
            pl.BlockSpec((B * I, H), lambda i: (0, 0)),
            pl.BlockSpec((_BN, H), lambda i: (i, 0)),
            pl.BlockSpec((1, N), lambda i: (0, 0)),
        ],
        out_specs=pl.BlockSpec((B * I, N), lambda i: (0, 0)),
        out_shape=jax.ShapeDtypeStruct((B * I, N), jnp.float32),
    )(x2, table_w0, b2)
    return out.reshape(B, I, N)


# speedup vs baseline: 1.0100x; 1.0037x over previous
import jax
import jax.numpy as jnp
from jax.experimental import pallas as pl


_BN = 4096


def _body(x_ref, w_ref, b_ref, o_ref):
    i = pl.program_id(0)
    nb = b_ref.shape[1] // _BN
    sl = pl.ds(i * _BN, _BN)

    @pl.when(i < nb)
    def _():
        o_ref[:, sl] = jax.lax.dot_general(
            x_ref[...], w_ref[...],
            dimension_numbers=(((1,), (1,)), ((), ())),
            preferred_element_type=jnp.float32,
        ) + b_ref[:, sl]

    if b_ref.shape[1] % _BN:
        tail = b_ref.shape[1] - nb * _BN
        tsl = pl.ds(nb * _BN, tail)

        @pl.when(i == nb)
        def _():
            o_ref[:, tsl] = jax.lax.dot_general(
                x_ref[...], w_ref[pl.ds(0, tail), :],
                dimension_numbers=(((1,), (1,)), ((), ())),
                preferred_element_type=jnp.float32,
            ) + b_ref[:, tsl]


def kernel(x, tgt, table_w0, table_b0):
    del tgt
    B, I, H = x.shape
    N = table_w0.shape[0]
    x2 = x.reshape(B * I, H)
    b2 = table_b0.reshape(1, N)
    out = pl.pallas_call(
        _body,
        grid=(pl.cdiv(N, _BN),),
        in_specs=[
            pl.BlockSpec((B * I, H), lambda i: (0, 0)),
            pl.BlockSpec((_BN, H), lambda i: (i, 0)),
            pl.BlockSpec((1, N), lambda i: (0, 0)),
        ],
        out_specs=pl.BlockSpec((B * I, N), lambda i: (0, 0)),
        out_shape=jax.ShapeDtypeStruct((B * I, N), jnp.float32),
    )(x2, table_w0, b2)
    return out.reshape(B, I, N)
